# baseline (speedup 1.0000x reference)
import functools

import jax
import jax.numpy as jnp
from jax import lax
from jax.experimental import pallas as pl
from jax.experimental.pallas import tpu as pltpu
from jax.experimental.pallas import tpu_sc as plsc

D = 128
NC, NS = 2, 16
NTILES = NC * NS
CHUNK = 128
RB = 1024


def _cdiv(a, b):
    return (a + b - 1) // b


def _score_body(x_ref, p_ref, s_ref):
    p = p_ref[...]
    nrm = jnp.sqrt(jnp.sum(p * p)) + 1e-16
    s_ref[...] = jnp.dot(x_ref[...], p) / nrm


def _score(x, p):
    n = x.shape[0]
    return pl.pallas_call(
        _score_body,
        out_shape=jax.ShapeDtypeStruct((n, 1), jnp.float32),
    )(x, p.reshape(D, 1))


def _deg_body(np_pad, n, cpt, dst3, ew3, x_hbm, idx_hbm, degp, xg,
              deg_sh, dst_v, ew_v, zb_v, idx_v, xg_v, sem):
    c = lax.axis_index("c")
    s = lax.axis_index("s")
    g = c * NS + s
    stripe = np_pad // NS
    def zb(i, carry):
        zb_v[pl.ds(i * 16, 16)] = jnp.zeros((16,), jnp.float32)
        return carry
    lax.fori_loop(0, stripe // 16, zb, 0)
    pltpu.sync_copy(zb_v, deg_sh.at[pl.ds(s * stripe, stripe)])
    plsc.subcore_barrier()
    pltpu.sync_copy(dst3.at[g], dst_v)
    pltpu.sync_copy(ew3.at[g], ew_v)
    def body(j, carry):
        pltpu.sync_copy(ew_v.at[j], deg_sh.at[dst_v.at[j]], add=True)
        return carry
    lax.fori_loop(0, cpt, body, 0)
    @pl.when(jnp.logical_and(c == 0, s == 1))
    def _():
        pltpu.sync_copy(idx_hbm, idx_v)
        pltpu.async_copy(x_hbm.at[idx_v], xg_v, sem).wait()
        pltpu.sync_copy(xg_v, xg)
    plsc.subcore_barrier()
    @pl.when(s == 0)
    def _():
        pltpu.sync_copy(deg_sh, degp.at[c])


def _deg(dst3, ew3, x, idx, np_pad, n):
    cpt = dst3.shape[1]
    mesh = plsc.VectorSubcoreMesh(core_axis_name="c", subcore_axis_name="s",
                                  num_cores=NC, num_subcores=NS)
    return pl.kernel(
        functools.partial(_deg_body, np_pad, n, cpt),
        out_type=(jax.ShapeDtypeStruct((NC, np_pad), jnp.float32),
                  jax.ShapeDtypeStruct((D, D), jnp.float32)),
        mesh=mesh,
        scratch_types=[
            pltpu.VMEM_SHARED((np_pad,), jnp.float32),
            pltpu.VMEM((cpt, CHUNK), jnp.int32),
            pltpu.VMEM((cpt, CHUNK), jnp.float32),
            pltpu.VMEM((np_pad // NS,), jnp.float32),
            pltpu.VMEM((D,), jnp.int32),
            pltpu.VMEM((D, D), jnp.float32),
            pltpu.SemaphoreType.DMA,
        ],
    )(dst3, ew3, x, idx)


def _dense_body(x_ref, xg_ref, tv_ref, h_ref, wih_ref, whh_ref,
                bih_ref, bhh_ref, dpt_ref, y_ref):
    xt = xg_ref[...] * jnp.tanh(tv_ref[...])
    gi = lax.dot_general(xt, wih_ref[...],
                         (((1,), (1,)), ((), ()))) + bih_ref[...]
    gh = lax.dot_general(h_ref[...], whh_ref[...],
                         (((1,), (1,)), ((), ()))) + bhh_ref[...]
    r = jax.nn.sigmoid(gi[:, :D] + gh[:, :D])
    z = jax.nn.sigmoid(gi[:, D:2 * D] + gh[:, D:2 * D])
    ncand = jnp.tanh(gi[:, 2 * D:] + r * gh[:, 2 * D:])
    W = (1.0 - z) * ncand + z * h_ref[...]
    xw = jnp.dot(x_ref[...], W)
    dp = dpt_ref[...]
    dinv = lax.rsqrt(dp[:, 0:1] + dp[:, 1:2] + 1.0)
    y_ref[...] = xw * dinv


def _dense(x, xg, vals, h, w_ih, w_hh, b_ih, b_hh, degp_t):
    n = x.shape[0]
    grid = _cdiv(n, RB)
    full = lambda shape: pl.BlockSpec(shape, lambda i: (0,) * len(shape))
    return pl.pallas_call(
        _dense_body,
        grid=(grid,),
        in_specs=[
            pl.BlockSpec((RB, D), lambda i: (i, 0)),
            full((D, D)),
            full((D, 1)),
            full((D, D)),
            full((3 * D, D)),
            full((3 * D, D)),
            full((1, 3 * D)),
            full((1, 3 * D)),
            pl.BlockSpec((RB, 2), lambda i: (i, 0)),
        ],
        out_specs=pl.BlockSpec((RB, D), lambda i: (i, 0)),
        out_shape=jax.ShapeDtypeStruct((n, D), jnp.float32),
    )(x, xg, vals.reshape(D, 1), h, w_ih, w_hh,
      b_ih.reshape(1, 3 * D), b_hh.reshape(1, 3 * D), degp_t)


def _msg_body(nz, cpt, idx3, ew4, y_hbm, zp,
              z_sh, stage, stw0, stw1, stw2, b0, b1, b2,
              gs0, gs1, gs2, ss0, ss1, ss2, stsem):
    c = lax.axis_index("c")
    s = lax.axis_index("s")
    g = c * NS + s
    stripe = (nz // (8 * NS)) * 8
    extra = nz - NS * stripe
    start = s * stripe
    bufs = (b0, b1, b2)
    stws = (stw0, stw1, stw2)
    gsems = (gs0, gs1, gs2)
    ssems = (ss0, ss1, ss2)

    @pl.when(c == 0)
    def _():
        pltpu.sync_copy(y_hbm.at[pl.ds(start, stripe)],
                        z_sh.at[pl.ds(start, stripe)])
        if extra:
            @pl.when(s == NS - 1)
            def _():
                pltpu.sync_copy(y_hbm.at[pl.ds(NS * stripe, extra)],
                                z_sh.at[pl.ds(NS * stripe, extra)])
    @pl.when(c != 0)
    def _():
        def zb(i, carry):
            for q in range(D // 16):
                b0[i, pl.ds(q * 16, 16)] = jnp.zeros((16,), jnp.float32)
            return carry
        lax.fori_loop(0, CHUNK, zb, 0)
        for k in range(stripe // CHUNK):
            pltpu.sync_copy(b0, z_sh.at[pl.ds(start + k * CHUNK, CHUNK)])
        rem = stripe % CHUNK
        if rem:
            pltpu.sync_copy(
                b0.at[pl.ds(0, rem)],
                z_sh.at[pl.ds(start + (stripe // CHUNK) * CHUNK, rem)])
        if extra:
            @pl.when(s == NS - 1)
            def _():
                pltpu.sync_copy(b0.at[pl.ds(0, extra)],
                                z_sh.at[pl.ds(NS * stripe, extra)])
    plsc.subcore_barrier()

    def scale(t):
        buf = bufs[t]
        stw_t = stws[t]
        def scale16(g16, c2):
            ew16 = stw_t[0, pl.ds(g16 * 16, 16)]
            for lane in range(16):
                w16 = jnp.take_along_axis(
                    ew16, jnp.full((16,), lane, jnp.int32), axis=0)
                e = g16 * 16 + lane
                for q in range(D // 16):
                    sl = pl.ds(q * 16, 16)
                    buf[e, sl] = buf[e, sl] * w16
            return c2
        lax.fori_loop(0, CHUNK // 16, scale16, 0)

    def stage_start(t, j):
        pltpu.async_copy(idx3.at[g, j], stage.at[t], stsem)
        pltpu.async_copy(ew4.at[g, j], stws[t], stsem)

    def stage_wait(t, j):
        pltpu.make_async_copy(idx3.at[g, j], stage.at[t], stsem).wait()
        pltpu.make_async_copy(ew4.at[g, j], stws[t], stsem).wait()

    def gather_start(t):
        pltpu.async_copy(y_hbm.at[stage.at[t, 0]], bufs[t], gsems[t])

    def wait_gather(t):
        pltpu.make_async_copy(y_hbm.at[stage.at[t, 0]], bufs[t],
                              gsems[t]).wait()

    def start_scatter(t):
        pltpu.async_copy(bufs[t], z_sh.at[stage.at[t, 1]], ssems[t],
                         add=True)

    def wait_scatter(t):
        pltpu.make_async_copy(bufs[t], z_sh.at[stage.at[t, 1]],
                              ssems[t]).wait()

    for t in (0, 1):
        stage_start(t, t)
        stage_wait(t, t)
        gather_start(t)
    wait_gather(0)
    stage_start(2, 2)
    scale(0)
    start_scatter(0)
    stage_wait(2, 2)
    gather_start(2)
    for t in (1, 2):
        tn = (t + 2) % 3
        wait_gather(t)
        wait_scatter(tn)
        stage_start(tn, t + 2)
        scale(t)
        start_scatter(t)
        stage_wait(tn, t + 2)
        gather_start(tn)
    def body(gg, carry):
        for t in range(3):
            j = 3 * gg + t
            tn = (t + 2) % 3
            wait_gather(t)
            wait_scatter(tn)
            stage_start(tn, j + 2)
            scale(t)
            start_scatter(t)
            stage_wait(tn, j + 2)
            gather_start(tn)
        return carry
    lax.fori_loop(1, cpt // 3 - 1, body, 0)
    for t in range(3):
        j = cpt - 3 + t
        tn = (t + 2) % 3
        wait_gather(t)
        wait_scatter(tn)
        if t == 0:
            stage_start(tn, j + 2)
        scale(t)
        start_scatter(t)
        if t == 0:
            stage_wait(tn, j + 2)
            gather_start(tn)
    wait_scatter((cpt - 1) % 3)

    plsc.subcore_barrier()
    pltpu.sync_copy(z_sh.at[pl.ds(start, stripe)],
                    zp.at[c, pl.ds(start, stripe)])
    if extra:
        @pl.when(s == NS - 1)
        def _():
            pltpu.sync_copy(z_sh.at[pl.ds(NS * stripe, extra)],
                            zp.at[c, pl.ds(NS * stripe, extra)])


def _msg(idx3, ew4, y, nz):
    cpt = idx3.shape[1]
    mesh = plsc.VectorSubcoreMesh(core_axis_name="c", subcore_axis_name="s",
                                  num_cores=NC, num_subcores=NS)
    return pl.kernel(
        functools.partial(_msg_body, nz, cpt),
        out_type=jax.ShapeDtypeStruct((NC, nz, D), jnp.float32),
        mesh=mesh,
        scratch_types=[
            pltpu.VMEM_SHARED((nz, D), jnp.float32),
            pltpu.VMEM((3, 2, CHUNK), jnp.int32),
            pltpu.VMEM((1, CHUNK), jnp.float32),
            pltpu.VMEM((1, CHUNK), jnp.float32),
            pltpu.VMEM((1, CHUNK), jnp.float32),
            pltpu.VMEM((CHUNK, D), jnp.float32),
            pltpu.VMEM((CHUNK, D), jnp.float32),
            pltpu.VMEM((CHUNK, D), jnp.float32),
            pltpu.SemaphoreType.DMA,
            pltpu.SemaphoreType.DMA,
            pltpu.SemaphoreType.DMA,
            pltpu.SemaphoreType.DMA,
            pltpu.SemaphoreType.DMA,
            pltpu.SemaphoreType.DMA,
            pltpu.SemaphoreType.DMA,
        ],
    )(idx3, ew4, y)


def _head_body(zp_ref, dpt_ref, lw_ref, lb_ref, hn_ref, lg_ref):
    dp = dpt_ref[...]
    dinv = lax.rsqrt(dp[:, 0:1] + dp[:, 1:2] + 1.0)
    hn = (zp_ref[0] + zp_ref[1]) * dinv
    hn_ref[...] = hn
    lg_ref[...] = jnp.dot(hn, lw_ref[...]) + lb_ref[...]


def _head(zp, degp_t, lin_w, lin_b):
    n = zp.shape[1]
    grid = _cdiv(n, RB)
    return pl.pallas_call(
        _head_body,
        grid=(grid,),
        in_specs=[
            pl.BlockSpec((NC, RB, D), lambda i: (0, i, 0)),
            pl.BlockSpec((RB, 2), lambda i: (i, 0)),
            pl.BlockSpec((D, 2), lambda i: (0, 0)),
            pl.BlockSpec((1, 2), lambda i: (0, 0)),
        ],
        out_specs=[
            pl.BlockSpec((RB, D), lambda i: (i, 0)),
            pl.BlockSpec((RB, 2), lambda i: (i, 0)),
        ],
        out_shape=[
            jax.ShapeDtypeStruct((n, D), jnp.float32),
            jax.ShapeDtypeStruct((n, 2), jnp.float32),
        ],
    )(zp, degp_t, lin_w, lin_b.reshape(1, 2))


def kernel(x, edge_index, edge_weight, h, p, w_ih, w_hh, b_ih, b_hh,
           lin_w, lin_b):
    n = x.shape[0]
    e = edge_weight.shape[0]
    np_pad = _cdiv(n, RB) * RB
    nz = _cdiv(n, NS) * NS
    cpt = max(2, _cdiv(_cdiv(e, NTILES * CHUNK), 3)) * 3
    e_pad = NTILES * cpt * CHUNK

    src = edge_index[0]
    dst = edge_index[1]
    npad = e_pad - e
    pad_idx = jnp.arange(npad, dtype=jnp.int32) % n
    src3 = jnp.concatenate([src, pad_idx]).reshape(NTILES, cpt, CHUNK)
    dst3 = jnp.concatenate([dst, pad_idx]).reshape(NTILES, cpt, CHUNK)
    ew_p = jnp.concatenate([edge_weight, jnp.zeros((npad,), jnp.float32)])
    ew3 = ew_p.reshape(NTILES, cpt, CHUNK)
    idx3 = jnp.stack([src3, dst3], axis=2)

    score = _score(x, p)[:, 0]
    vals, idx = lax.top_k(score, D)

    degp, xg = _deg(dst3, ew3, x, idx.astype(jnp.int32), np_pad, n)
    degp_t = degp.T

    y = _dense(x, xg, vals, h, w_ih, w_hh, b_ih, b_hh, degp_t)
    zp = _msg(idx3, ew_p.reshape(NTILES, cpt, 1, CHUNK), y, nz)
    h_nodes, logits = _head(zp, degp_t, lin_w, lin_b)
    return (logits, h_nodes)


# speedup vs baseline: 1.1021x; 1.1021x over previous
import functools

import jax
import jax.numpy as jnp
from jax import lax
from jax.experimental import pallas as pl
from jax.experimental.pallas import tpu as pltpu
from jax.experimental.pallas import tpu_sc as plsc

D = 128
NC, NS = 2, 16
NTILES = NC * NS
CHUNK = 128
RB = 1024


def _cdiv(a, b):
    return (a + b - 1) // b


def _score_body(x_ref, p_ref, s_ref):
    p = p_ref[...]
    nrm = jnp.sqrt(jnp.sum(p * p)) + 1e-16
    s_ref[...] = jnp.dot(x_ref[...], p) / nrm


def _score(x, p):
    n = x.shape[0]
    return pl.pallas_call(
        _score_body,
        out_shape=jax.ShapeDtypeStruct((n, 1), jnp.float32),
    )(x, p.reshape(D, 1))


def _deg_body(np_pad, n, cpt, idx3, ew3, x_hbm, idx_hbm, degp, xg,
              deg_sh, dst_v, ew_v, zb_v, idx_v, xg_v, sem):
    c = lax.axis_index("c")
    s = lax.axis_index("s")
    g = c * NS + s
    stripe = np_pad // NS
    def zb(i, carry):
        zb_v[pl.ds(i * 16, 16)] = jnp.zeros((16,), jnp.float32)
        return carry
    lax.fori_loop(0, stripe // 16, zb, 0)
    pltpu.sync_copy(zb_v, deg_sh.at[pl.ds(s * stripe, stripe)])
    plsc.subcore_barrier()
    pltpu.sync_copy(idx3.at[g], dst_v)
    pltpu.sync_copy(ew3.at[g], ew_v)
    def body(j, carry):
        pltpu.sync_copy(ew_v.at[j], deg_sh.at[dst_v.at[j, 1]], add=True)
        return carry
    lax.fori_loop(0, cpt, body, 0)
    @pl.when(jnp.logical_and(c == 0, s == 1))
    def _():
        pltpu.sync_copy(idx_hbm, idx_v)
        pltpu.async_copy(x_hbm.at[idx_v], xg_v, sem).wait()
        pltpu.sync_copy(xg_v, xg)
    plsc.subcore_barrier()
    @pl.when(s == 0)
    def _():
        pltpu.sync_copy(deg_sh, degp.at[c])


def _deg(idx3, ew3, x, idx, np_pad, n):
    cpt = idx3.shape[1]
    mesh = plsc.VectorSubcoreMesh(core_axis_name="c", subcore_axis_name="s",
                                  num_cores=NC, num_subcores=NS)
    return pl.kernel(
        functools.partial(_deg_body, np_pad, n, cpt),
        out_type=(jax.ShapeDtypeStruct((NC, np_pad), jnp.float32),
                  jax.ShapeDtypeStruct((D, D), jnp.float32)),
        mesh=mesh,
        scratch_types=[
            pltpu.VMEM_SHARED((np_pad,), jnp.float32),
            pltpu.VMEM((cpt, 2, CHUNK), jnp.int32),
            pltpu.VMEM((cpt, CHUNK), jnp.float32),
            pltpu.VMEM((np_pad // NS,), jnp.float32),
            pltpu.VMEM((D,), jnp.int32),
            pltpu.VMEM((D, D), jnp.float32),
            pltpu.SemaphoreType.DMA,
        ],
    )(idx3, ew3, x, idx)


def _dense_body(x_ref, xg_ref, tv_ref, h_ref, wih_ref, whh_ref,
                bih_ref, bhh_ref, dp_ref, y_ref, w_ref):
    @pl.when(pl.program_id(0) == 0)
    def _():
        xt = xg_ref[...] * jnp.tanh(tv_ref[...])
        gi = lax.dot_general(xt, wih_ref[...],
                             (((1,), (1,)), ((), ()))) + bih_ref[...]
        gh = lax.dot_general(h_ref[...], whh_ref[...],
                             (((1,), (1,)), ((), ()))) + bhh_ref[...]
        r = jax.nn.sigmoid(gi[:, :D] + gh[:, :D])
        z = jax.nn.sigmoid(gi[:, D:2 * D] + gh[:, D:2 * D])
        ncand = jnp.tanh(gi[:, 2 * D:] + r * gh[:, 2 * D:])
        w_ref[...] = (1.0 - z) * ncand + z * h_ref[...]
    xw = jnp.dot(x_ref[...], w_ref[...])
    ones = jnp.ones((2, 1), jnp.float32)
    deg = lax.dot_general(dp_ref[...], ones,
                          (((0,), (0,)), ((), ()))) + 1.0
    y_ref[...] = xw * lax.rsqrt(deg)


def _dense(x, xg, vals, h, w_ih, w_hh, b_ih, b_hh, degp):
    n = x.shape[0]
    grid = _cdiv(n, RB)
    full = lambda shape: pl.BlockSpec(shape, lambda i: (0,) * len(shape))
    return pl.pallas_call(
        _dense_body,
        grid=(grid,),
        in_specs=[
            pl.BlockSpec((RB, D), lambda i: (i, 0)),
            full((D, D)),
            full((D, 1)),
            full((D, D)),
            full((3 * D, D)),
            full((3 * D, D)),
            full((1, 3 * D)),
            full((1, 3 * D)),
            pl.BlockSpec((2, RB), lambda i: (0, i)),
        ],
        out_specs=pl.BlockSpec((RB, D), lambda i: (i, 0)),
        out_shape=jax.ShapeDtypeStruct((n, D), jnp.float32),
        scratch_shapes=[pltpu.VMEM((D, D), jnp.float32)],
    )(x, xg, vals.reshape(D, 1), h, w_ih, w_hh,
      b_ih.reshape(1, 3 * D), b_hh.reshape(1, 3 * D), degp)


def _msg_body(nz, cpt, idx3, ew4, y_hbm, zp,
              z_sh, stage, stw0, stw1, stw2, b0, b1, b2,
              gs0, gs1, gs2, ss0, ss1, ss2, stsem):
    c = lax.axis_index("c")
    s = lax.axis_index("s")
    g = c * NS + s
    stripe = (nz // (8 * NS)) * 8
    extra = nz - NS * stripe
    start = s * stripe
    bufs = (b0, b1, b2)
    stws = (stw0, stw1, stw2)
    gsems = (gs0, gs1, gs2)
    ssems = (ss0, ss1, ss2)

    @pl.when(c == 0)
    def _():
        pltpu.sync_copy(y_hbm.at[pl.ds(start, stripe)],
                        z_sh.at[pl.ds(start, stripe)])
        if extra:
            @pl.when(s == NS - 1)
            def _():
                pltpu.sync_copy(y_hbm.at[pl.ds(NS * stripe, extra)],
                                z_sh.at[pl.ds(NS * stripe, extra)])
    @pl.when(c != 0)
    def _():
        def zb(i, carry):
            for q in range(D // 16):
                b0[i, pl.ds(q * 16, 16)] = jnp.zeros((16,), jnp.float32)
            return carry
        lax.fori_loop(0, CHUNK, zb, 0)
        for k in range(stripe // CHUNK):
            pltpu.sync_copy(b0, z_sh.at[pl.ds(start + k * CHUNK, CHUNK)])
        rem = stripe % CHUNK
        if rem:
            pltpu.sync_copy(
                b0.at[pl.ds(0, rem)],
                z_sh.at[pl.ds(start + (stripe // CHUNK) * CHUNK, rem)])
        if extra:
            @pl.when(s == NS - 1)
            def _():
                pltpu.sync_copy(b0.at[pl.ds(0, extra)],
                                z_sh.at[pl.ds(NS * stripe, extra)])
    plsc.subcore_barrier()

    def scale(t):
        buf = bufs[t]
        stw_t = stws[t]
        def scale16(g16, c2):
            ew16 = stw_t[0, pl.ds(g16 * 16, 16)]
            for lane in range(16):
                w16 = jnp.take_along_axis(
                    ew16, jnp.full((16,), lane, jnp.int32), axis=0)
                e = g16 * 16 + lane
                for q in range(D // 16):
                    sl = pl.ds(q * 16, 16)
                    buf[e, sl] = buf[e, sl] * w16
            return c2
        lax.fori_loop(0, CHUNK // 16, scale16, 0)

    def stage_start(t, j):
        pltpu.async_copy(idx3.at[g, j], stage.at[t], stsem)
        pltpu.async_copy(ew4.at[g, j], stws[t], stsem)

    def stage_wait(t, j):
        pltpu.make_async_copy(idx3.at[g, j], stage.at[t], stsem).wait()
        pltpu.make_async_copy(ew4.at[g, j], stws[t], stsem).wait()

    def gather_start(t):
        pltpu.async_copy(y_hbm.at[stage.at[t, 0]], bufs[t], gsems[t])

    def wait_gather(t):
        pltpu.make_async_copy(y_hbm.at[stage.at[t, 0]], bufs[t],
                              gsems[t]).wait()

    def start_scatter(t):
        pltpu.async_copy(bufs[t], z_sh.at[stage.at[t, 1]], ssems[t],
                         add=True)

    def wait_scatter(t):
        pltpu.make_async_copy(bufs[t], z_sh.at[stage.at[t, 1]],
                              ssems[t]).wait()

    for t in (0, 1):
        stage_start(t, t)
        stage_wait(t, t)
        gather_start(t)
    wait_gather(0)
    stage_start(2, 2)
    scale(0)
    start_scatter(0)
    stage_wait(2, 2)
    gather_start(2)
    for t in (1, 2):
        tn = (t + 2) % 3
        wait_gather(t)
        wait_scatter(tn)
        stage_start(tn, t + 2)
        scale(t)
        start_scatter(t)
        stage_wait(tn, t + 2)
        gather_start(tn)
    def body(gg, carry):
        for t in range(3):
            j = 3 * gg + t
            tn = (t + 2) % 3
            wait_gather(t)
            wait_scatter(tn)
            stage_start(tn, j + 2)
            scale(t)
            start_scatter(t)
            stage_wait(tn, j + 2)
            gather_start(tn)
        return carry
    lax.fori_loop(1, cpt // 3 - 1, body, 0)
    for t in range(3):
        j = cpt - 3 + t
        tn = (t + 2) % 3
        wait_gather(t)
        wait_scatter(tn)
        if t == 0:
            stage_start(tn, j + 2)
        scale(t)
        start_scatter(t)
        if t == 0:
            stage_wait(tn, j + 2)
            gather_start(tn)
    wait_scatter((cpt - 1) % 3)

    plsc.subcore_barrier()
    pltpu.sync_copy(z_sh.at[pl.ds(start, stripe)],
                    zp.at[c, pl.ds(start, stripe)])
    if extra:
        @pl.when(s == NS - 1)
        def _():
            pltpu.sync_copy(z_sh.at[pl.ds(NS * stripe, extra)],
                            zp.at[c, pl.ds(NS * stripe, extra)])


def _msg(idx3, ew4, y, nz):
    cpt = idx3.shape[1]
    mesh = plsc.VectorSubcoreMesh(core_axis_name="c", subcore_axis_name="s",
                                  num_cores=NC, num_subcores=NS)
    return pl.kernel(
        functools.partial(_msg_body, nz, cpt),
        out_type=jax.ShapeDtypeStruct((NC, nz, D), jnp.float32),
        mesh=mesh,
        scratch_types=[
            pltpu.VMEM_SHARED((nz, D), jnp.float32),
            pltpu.VMEM((3, 2, CHUNK), jnp.int32),
            pltpu.VMEM((1, CHUNK), jnp.float32),
            pltpu.VMEM((1, CHUNK), jnp.float32),
            pltpu.VMEM((1, CHUNK), jnp.float32),
            pltpu.VMEM((CHUNK, D), jnp.float32),
            pltpu.VMEM((CHUNK, D), jnp.float32),
            pltpu.VMEM((CHUNK, D), jnp.float32),
            pltpu.SemaphoreType.DMA,
            pltpu.SemaphoreType.DMA,
            pltpu.SemaphoreType.DMA,
            pltpu.SemaphoreType.DMA,
            pltpu.SemaphoreType.DMA,
            pltpu.SemaphoreType.DMA,
            pltpu.SemaphoreType.DMA,
        ],
    )(idx3, ew4, y)


def _head_body(zp_ref, dp_ref, lw_ref, lb_ref, hn_ref, lg_ref):
    ones = jnp.ones((2, 1), jnp.float32)
    deg = lax.dot_general(dp_ref[...], ones,
                          (((0,), (0,)), ((), ()))) + 1.0
    hn = (zp_ref[0] + zp_ref[1]) * lax.rsqrt(deg)
    hn_ref[...] = hn
    lg_ref[...] = lax.dot_general(lw_ref[...], hn,
                                  (((0,), (1,)), ((), ()))) + lb_ref[...]


def _head(zp, degp, lin_w, lin_b):
    n = zp.shape[1]
    grid = _cdiv(n, RB)
    return pl.pallas_call(
        _head_body,
        grid=(grid,),
        in_specs=[
            pl.BlockSpec((NC, RB, D), lambda i: (0, i, 0)),
            pl.BlockSpec((2, RB), lambda i: (0, i)),
            pl.BlockSpec((D, 2), lambda i: (0, 0)),
            pl.BlockSpec((2, 1), lambda i: (0, 0)),
        ],
        out_specs=[
            pl.BlockSpec((RB, D), lambda i: (i, 0)),
            pl.BlockSpec((2, RB), lambda i: (0, i)),
        ],
        out_shape=[
            jax.ShapeDtypeStruct((n, D), jnp.float32),
            jax.ShapeDtypeStruct((2, n), jnp.float32),
        ],
    )(zp, degp, lin_w, lin_b.reshape(2, 1))


def kernel(x, edge_index, edge_weight, h, p, w_ih, w_hh, b_ih, b_hh,
           lin_w, lin_b):
    n = x.shape[0]
    e = edge_weight.shape[0]
    np_pad = _cdiv(n, RB) * RB
    nz = _cdiv(n, NS) * NS
    cpt = max(2, _cdiv(_cdiv(e, NTILES * CHUNK), 3)) * 3
    e_pad = NTILES * cpt * CHUNK

    npad = e_pad - e
    pad_idx = jnp.arange(npad, dtype=jnp.int32) % n
    ei_pad = jnp.concatenate(
        [edge_index, jnp.broadcast_to(pad_idx, (2, npad))], axis=1)
    idx3 = ei_pad.reshape(2, NTILES, cpt, CHUNK).transpose(1, 2, 0, 3)
    ew_p = jnp.concatenate([edge_weight, jnp.zeros((npad,), jnp.float32)])
    ew3 = ew_p.reshape(NTILES, cpt, CHUNK)

    score = _score(x, p)[:, 0]
    vals, idx = lax.top_k(score, D)

    degp, xg = _deg(idx3, ew3, x, idx.astype(jnp.int32), np_pad, n)

    y = _dense(x, xg, vals, h, w_ih, w_hh, b_ih, b_hh, degp)
    zp = _msg(idx3, ew_p.reshape(NTILES, cpt, 1, CHUNK), y, nz)
    h_nodes, logits_t = _head(zp, degp, lin_w, lin_b)
    return (logits_t.T, h_nodes)
